# speedup vs baseline: 6.6442x; 6.6442x over previous
import functools

import jax
import jax.numpy as jnp
from jax.experimental import pallas as pl

BLK = 256
LATS = 32
KSEL = 8
THRESH = 0.1
EPS = 1e-08


def _fused_kernel(x_ref, w_ref, b_ref, d_ref, o_ref):
    blk = x_ref.shape[0]
    zt = jax.lax.dot_general(
        w_ref[...], x_ref[...], (((1,), (1,)), ((), ())),
        preferred_element_type=jnp.float32)
    zt = jax.nn.sigmoid(zt + b_ref[...])

    iota = jax.lax.broadcasted_iota(jnp.int32, (LATS, blk), 0)
    zw = zt
    mask = jnp.zeros((LATS, blk), jnp.bool_)
    for _ in range(KSEL):
        mx = jnp.max(zw, axis=0, keepdims=True)
        am = jnp.min(jnp.where(zw == mx, iota, LATS), axis=0, keepdims=True)
        sel = iota == am
        mask = mask | sel
        zw = jnp.where(sel, -1.0, zw)
    zm = jnp.where(mask & (zt > THRESH), zt, 0.0)

    zb = zm.T
    dc = jnp.clip(d_ref[...], 0.0, 1.0)
    acc = jnp.full((blk, dc.shape[1]), 1.0 + EPS, jnp.float32)
    for l in range(LATS):
        vl = jax.lax.slice_in_dim(zb, l, l + 1, axis=1)
        dl = jax.lax.slice_in_dim(dc, l, l + 1, axis=0)
        acc = acc * ((1.0 + EPS) - vl * dl)
    o_ref[...] = jnp.clip(1.0 - acc, 1e-07, 1.0 - 1e-07)


@jax.jit
def kernel(x, W, b, D):
    batch, din = x.shape
    grid = (batch // BLK,)
    return pl.pallas_call(
        _fused_kernel,
        grid=grid,
        in_specs=[
            pl.BlockSpec((BLK, din), lambda i: (i, 0)),
            pl.BlockSpec((LATS, din), lambda i: (0, 0)),
            pl.BlockSpec((LATS, 1), lambda i: (0, 0)),
            pl.BlockSpec((LATS, din), lambda i: (0, 0)),
        ],
        out_specs=pl.BlockSpec((BLK, din), lambda i: (i, 0)),
        out_shape=jax.ShapeDtypeStruct((batch, din), jnp.float32),
    )(x, W, b.reshape(LATS, 1), D)
